# baseline (device time: 24478 ns/iter reference)
import jax
import jax.numpy as jnp
from jax import lax
from jax.experimental import pallas as pl
from jax.experimental.pallas import tpu as pltpu

F32 = jnp.float32
BF16 = jnp.bfloat16


def kernel(x, router, W1, W2):
    t_per, d = x.shape
    e_per = W1.shape[0]
    assert router.shape == (d, e_per)

    def body(x_ref, r_ref, w1_ref, w2_ref, out_ref,
             xs_send, xs_recv, r_recv, wt_send, wt_recv,
             ps_send, ps_recv, send_sems, recv_sems):
        my_x = lax.axis_index("x")
        my_y = lax.axis_index("y")
        peer = (my_x, 1 - my_y)

        barrier = pltpu.get_barrier_semaphore()
        pl.semaphore_signal(barrier, inc=1, device_id=peer,
                            device_id_type=pl.DeviceIdType.MESH)
        pl.semaphore_wait(barrier, 1)

        def exchange(slot, src, dst):
            rdma = pltpu.make_async_remote_copy(
                src_ref=src, dst_ref=dst,
                send_sem=send_sems.at[slot], recv_sem=recv_sems.at[slot],
                device_id=peer, device_id_type=pl.DeviceIdType.MESH)
            rdma.start()
            return rdma

        xs_send[...] = x_ref[...].astype(BF16)
        r_rdma = exchange(0, r_ref, r_recv)
        x_rdma = exchange(1, xs_send, xs_recv)

        def gates(rhs):
            return lax.dot_general(
                x_ref[...], rhs, (((1,), (0,)), ((), ())),
                precision=lax.Precision.HIGHEST, preferred_element_type=F32)

        gl = gates(r_ref[...])
        r_rdma.wait_recv()
        gr = gates(r_recv[...])

        a1 = jnp.max(gl, axis=1, keepdims=True)
        a2 = jnp.min(gl, axis=1, keepdims=True)
        b1 = jnp.max(gr, axis=1, keepdims=True)
        b2 = jnp.min(gr, axis=1, keepdims=True)
        m1 = jnp.maximum(a1, b1)
        m2 = jnp.maximum(jnp.minimum(a1, b1), jnp.where(a1 > b1, a2, b2))
        denom = 1.0 + jnp.exp(m2 - m1)
        w_loc = jnp.where(gl >= m2, jnp.exp(gl - m1), 0.0) / denom
        wt_send[...] = jnp.where(gr >= m2, jnp.exp(gr - m1), 0.0) / denom
        w_rdma = exchange(2, wt_send, wt_recv)

        def run_experts(xb, wts):
            acc = jnp.zeros((xb.shape[0], d), F32)
            for e in range(e_per):
                h = lax.dot_general(xb, w1_ref[e].astype(BF16),
                                    (((1,), (0,)), ((), ())),
                                    preferred_element_type=F32)
                h = jnp.maximum(h, 0.0).astype(BF16)
                o = lax.dot_general(h, w2_ref[e].astype(BF16),
                                    (((1,), (0,)), ((), ())),
                                    preferred_element_type=F32)
                acc = acc + o * wts[:, e:e + 1]
            return acc

        acc_mine = run_experts(xs_send[...], w_loc)

        x_rdma.wait_recv()
        w_rdma.wait_recv()
        ps_send[...] = run_experts(xs_recv[...], wt_recv[...]).astype(BF16)
        p_rdma = exchange(3, ps_send, ps_recv)

        p_rdma.wait_recv()
        out_ref[...] = acc_mine + ps_recv[...].astype(F32)

        for rdma in (r_rdma, x_rdma, w_rdma, p_rdma):
            rdma.wait_send()

    return pl.pallas_call(
        body,
        out_shape=jax.ShapeDtypeStruct((t_per, d), F32),
        in_specs=[pl.BlockSpec(memory_space=pltpu.VMEM)] * 4,
        out_specs=pl.BlockSpec(memory_space=pltpu.VMEM),
        scratch_shapes=[
            pltpu.VMEM((t_per, d), BF16),
            pltpu.VMEM((t_per, d), BF16),
            pltpu.VMEM((d, e_per), F32),
            pltpu.VMEM((t_per, e_per), F32),
            pltpu.VMEM((t_per, e_per), F32),
            pltpu.VMEM((t_per, d), BF16),
            pltpu.VMEM((t_per, d), BF16),
            pltpu.SemaphoreType.DMA((4,)),
            pltpu.SemaphoreType.DMA((4,)),
        ],
        compiler_params=pltpu.CompilerParams(collective_id=0),
    )(x, router, W1, W2)
